# baseline (speedup 1.0000x reference)
import functools

import jax
import jax.numpy as jnp
import numpy as np
from jax import lax
from jax.experimental import pallas as pl
from jax.experimental.pallas import tpu as pltpu

_ROT_A = (13, 15, 26, 6)
_ROT_B = (17, 29, 16, 24)
_KS = (np.uint32(0), np.uint32(1), np.uint32(0x1BD11BDB))
_INJ = ((1, 2), (2, 0), (0, 1), (1, 2), (2, 0))


def _rotl(x, r):
    return (x << np.uint32(r)) | (x >> np.uint32(32 - r))


def _threefry_bits(i_u32):
    x1 = i_u32 + _KS[1]
    x0 = x1
    x1 = _rotl(x1, _ROT_A[0]) ^ x0
    for r in _ROT_A[1:]:
        x0 = x0 + x1
        x1 = _rotl(x1, r)
        x1 = x1 ^ x0
    x0 = x0 + _KS[1]
    x1 = x1 + np.uint32(0x1BD11BDB + 1)
    for g, rots in ((1, _ROT_B), (2, _ROT_A), (3, _ROT_B), (4, _ROT_A)):
        for r in rots:
            x0 = x0 + x1
            x1 = _rotl(x1, r)
            x1 = x1 ^ x0
        a, b = _INJ[g]
        if _KS[a]:
            x0 = x0 + _KS[a]
        x1 = x1 + (_KS[b] + np.uint32(g + 1))
    return x0 ^ x1


_NEG_LN2 = np.float32(-np.log(np.float32(2.0), dtype=np.float32))


def _sample_block(mask_ref, out_ref, base_ref, amax_ref, acid_ref,
                  *, ncols, blk_c, nblk):
    c = pl.program_id(0)
    nrows = mask_ref.shape[0]
    shape = (nrows, blk_c)

    @pl.when(c == 0)
    def _init():
        row = lax.broadcasted_iota(jnp.uint32, shape, 0)
        lane_u = lax.broadcasted_iota(jnp.uint32, shape, 1)
        base_ref[...] = row * np.uint32(ncols) + lane_u
        amax_ref[...] = jnp.full(shape, -jnp.inf, jnp.float32)

    def _update(mask_tail):
        flat = base_ref[...] + (c * blk_c).astype(jnp.uint32)
        bits = _threefry_bits(flat)

        f = lax.bitcast_convert_type(
            (bits >> np.uint32(9)) | np.uint32(0x3F800000),
            jnp.float32) - np.float32(1.0)
        u = jnp.maximum(np.float32(1e-9), f + np.float32(1e-9))
        w = jnp.log2(u) * _NEG_LN2
        gumbel = jnp.log2(w) * _NEG_LN2
        val = jnp.log(mask_ref[...] + np.float32(1e-20)) + gumbel

        if mask_tail:
            lane = lax.broadcasted_iota(jnp.int32, shape, 1)
            val = jnp.where(lane < ncols - c * blk_c, val, -jnp.inf)

        take = val > amax_ref[...]
        amax_ref[...] = jnp.maximum(amax_ref[...], val)
        acid_ref[...] = jnp.where(take, c, acid_ref[...])

    @pl.when(c < nblk - 1)
    def _steady():
        _update(False)

    @pl.when(c == nblk - 1)
    def _final():
        _update(True)
        am = amax_ref[...]
        rowmax = jnp.max(am, axis=1)
        j = acid_ref[...] * blk_c + lax.broadcasted_iota(jnp.int32, shape, 1)
        sel = jnp.where(am == rowmax[:, None], j, np.int32(0x7FFFFFFF))
        out_ref[...] = jnp.min(sel, axis=1)


@jax.jit
def kernel(action_mask):
    nrows, ncols = action_mask.shape
    blk_c = 1024
    nblk = pl.cdiv(ncols, blk_c)
    return pl.pallas_call(
        functools.partial(_sample_block, ncols=ncols, blk_c=blk_c, nblk=nblk),
        grid=(nblk,),
        in_specs=[pl.BlockSpec((nrows, blk_c), lambda c: (0, c))],
        out_specs=pl.BlockSpec((nrows,), lambda c: (0,)),
        out_shape=jax.ShapeDtypeStruct((nrows,), jnp.int32),
        scratch_shapes=[pltpu.VMEM((nrows, blk_c), jnp.uint32),
                        pltpu.VMEM((nrows, blk_c), jnp.float32),
                        pltpu.VMEM((nrows, blk_c), jnp.int32)],
        compiler_params=pltpu.CompilerParams(
            dimension_semantics=("arbitrary",)),
    )(action_mask)


# speedup vs baseline: 1.9071x; 1.9071x over previous
import functools

import jax
import jax.numpy as jnp
import numpy as np
from jax import lax
from jax.experimental import pallas as pl
from jax.experimental.pallas import tpu as pltpu

_ROT_A = (13, 15, 26, 6)
_ROT_B = (17, 29, 16, 24)
_KS = (np.uint32(0), np.uint32(1), np.uint32(0x1BD11BDB))
_INJ = ((1, 2), (2, 0), (0, 1), (1, 2), (2, 0))


def _rotl(x, r):
    return (x << np.uint32(r)) | (x >> np.uint32(32 - r))


def _threefry_bits(i_u32):
    x1 = i_u32 + _KS[1]
    x0 = x1
    x1 = _rotl(x1, _ROT_A[0]) ^ x0
    for r in _ROT_A[1:]:
        x0 = x0 + x1
        x1 = _rotl(x1, r)
        x1 = x1 ^ x0
    x0 = x0 + _KS[1]
    x1 = x1 + np.uint32(0x1BD11BDB + 1)
    for g, rots in ((1, _ROT_B), (2, _ROT_A), (3, _ROT_B), (4, _ROT_A)):
        for r in rots:
            x0 = x0 + x1
            x1 = _rotl(x1, r)
            x1 = x1 ^ x0
        a, b = _INJ[g]
        if _KS[a]:
            x0 = x0 + _KS[a]
        x1 = x1 + (_KS[b] + np.uint32(g + 1))
    return x0 ^ x1


_NEG_LN2 = np.float32(-np.log(np.float32(2.0), dtype=np.float32))


def _sample_block(mask_ref, out_ref, base_ref, amax_ref, acid_ref,
                  *, ncols, blk_c, nblk):
    c = pl.program_id(0)
    nrows = mask_ref.shape[0]
    shape = (nrows, blk_c)

    @pl.when(c == 0)
    def _init():
        row = lax.broadcasted_iota(jnp.uint32, shape, 0)
        lane_u = lax.broadcasted_iota(jnp.uint32, shape, 1)
        base_ref[...] = row * np.uint32(ncols) + lane_u
        amax_ref[...] = jnp.full(shape, -jnp.inf, jnp.float32)

    flat = base_ref[...] + (c * blk_c).astype(jnp.uint32)
    bits = _threefry_bits(flat)

    f = lax.bitcast_convert_type(
        (bits >> np.uint32(9)) | np.uint32(0x3F800000),
        jnp.float32) - np.float32(1.0)
    u = jnp.maximum(np.float32(1e-9), f + np.float32(1e-9))
    w = jnp.log2(u) * _NEG_LN2
    gumbel = jnp.log2(w) * _NEG_LN2
    val = jnp.log(mask_ref[...] + np.float32(1e-20)) + gumbel

    lane = lax.broadcasted_iota(jnp.int32, shape, 1)
    val = jnp.where(lane < ncols - c * blk_c, val, -jnp.inf)

    take = val > amax_ref[...]
    amax_ref[...] = jnp.maximum(amax_ref[...], val)
    acid_ref[...] = jnp.where(take, c, acid_ref[...])

    @pl.when(c == nblk - 1)
    def _final():
        am = amax_ref[...]
        rowmax = jnp.max(am, axis=1)
        j = acid_ref[...] * blk_c + lax.broadcasted_iota(jnp.int32, shape, 1)
        sel = jnp.where(am == rowmax[:, None], j, np.int32(0x7FFFFFFF))
        out_ref[...] = jnp.min(sel, axis=1)


@jax.jit
def kernel(action_mask):
    nrows, ncols = action_mask.shape
    blk_c = 1024
    nblk = pl.cdiv(ncols, blk_c)
    return pl.pallas_call(
        functools.partial(_sample_block, ncols=ncols, blk_c=blk_c, nblk=nblk),
        grid=(nblk,),
        in_specs=[pl.BlockSpec((nrows, blk_c), lambda c: (0, c))],
        out_specs=pl.BlockSpec((nrows,), lambda c: (0,)),
        out_shape=jax.ShapeDtypeStruct((nrows,), jnp.int32),
        scratch_shapes=[pltpu.VMEM((nrows, blk_c), jnp.uint32),
                        pltpu.VMEM((nrows, blk_c), jnp.float32),
                        pltpu.VMEM((nrows, blk_c), jnp.int32)],
        compiler_params=pltpu.CompilerParams(
            dimension_semantics=("arbitrary",)),
    )(action_mask)


# speedup vs baseline: 1.9331x; 1.0136x over previous
import functools

import jax
import jax.numpy as jnp
import numpy as np
from jax import lax
from jax.experimental import pallas as pl
from jax.experimental.pallas import tpu as pltpu

_ROT_A = (13, 15, 26, 6)
_ROT_B = (17, 29, 16, 24)
_KS = (np.uint32(0), np.uint32(1), np.uint32(0x1BD11BDB))
_INJ = ((1, 2), (2, 0), (0, 1), (1, 2), (2, 0))


def _rotl(x, r):
    return (x << np.uint32(r)) | (x >> np.uint32(32 - r))


def _threefry_bits(i_u32):
    x1 = i_u32 + _KS[1]
    x0 = x1
    x1 = _rotl(x1, _ROT_A[0]) ^ x0
    for r in _ROT_A[1:]:
        x0 = x0 + x1
        x1 = _rotl(x1, r)
        x1 = x1 ^ x0
    x0 = x0 + _KS[1]
    x1 = x1 + np.uint32(0x1BD11BDB + 1)
    for g, rots in ((1, _ROT_B), (2, _ROT_A), (3, _ROT_B), (4, _ROT_A)):
        for r in rots:
            x0 = x0 + x1
            x1 = _rotl(x1, r)
            x1 = x1 ^ x0
        a, b = _INJ[g]
        if _KS[a]:
            x0 = x0 + _KS[a]
        x1 = x1 + (_KS[b] + np.uint32(g + 1))
    return x0 ^ x1


def _sample_block(mask_ref, out_ref, base_ref, amax_ref, acid_ref,
                  *, ncols, blk_c, nblk):
    c = pl.program_id(0)
    nrows = mask_ref.shape[0]
    shape = (nrows, blk_c)

    @pl.when(c == 0)
    def _init():
        row = lax.broadcasted_iota(jnp.uint32, shape, 0)
        lane_u = lax.broadcasted_iota(jnp.uint32, shape, 1)
        base_ref[...] = row * np.uint32(ncols) + lane_u
        amax_ref[...] = jnp.full(shape, -jnp.inf, jnp.float32)

    flat = base_ref[...] + (c * blk_c).astype(jnp.uint32)
    bits = _threefry_bits(flat)

    f = lax.bitcast_convert_type(
        (bits >> np.uint32(9)) | np.uint32(0x3F800000),
        jnp.float32) - np.float32(1.0)
    u = jnp.maximum(np.float32(1e-9), f + np.float32(1e-9))
    gumbel = -jnp.log(-jnp.log(u))
    val = jnp.log(mask_ref[...] + np.float32(1e-20)) + gumbel

    lane = lax.broadcasted_iota(jnp.int32, shape, 1)
    val = jnp.where(lane < ncols - c * blk_c, val, -jnp.inf)

    take = val > amax_ref[...]
    amax_ref[...] = jnp.maximum(amax_ref[...], val)
    acid_ref[...] = jnp.where(take, c, acid_ref[...])

    @pl.when(c == nblk - 1)
    def _final():
        am = amax_ref[...]
        rowmax = jnp.max(am, axis=1)
        j = acid_ref[...] * blk_c + lax.broadcasted_iota(jnp.int32, shape, 1)
        sel = jnp.where(am == rowmax[:, None], j, np.int32(0x7FFFFFFF))
        out_ref[...] = jnp.min(sel, axis=1)


@jax.jit
def kernel(action_mask):
    nrows, ncols = action_mask.shape
    blk_c = 1024
    nblk = pl.cdiv(ncols, blk_c)
    return pl.pallas_call(
        functools.partial(_sample_block, ncols=ncols, blk_c=blk_c, nblk=nblk),
        grid=(nblk,),
        in_specs=[pl.BlockSpec((nrows, blk_c), lambda c: (0, c))],
        out_specs=pl.BlockSpec((nrows,), lambda c: (0,)),
        out_shape=jax.ShapeDtypeStruct((nrows,), jnp.int32),
        scratch_shapes=[pltpu.VMEM((nrows, blk_c), jnp.uint32),
                        pltpu.VMEM((nrows, blk_c), jnp.float32),
                        pltpu.VMEM((nrows, blk_c), jnp.int32)],
        compiler_params=pltpu.CompilerParams(
            dimension_semantics=("arbitrary",)),
    )(action_mask)
